# Initial kernel scaffold; baseline (speedup 1.0000x reference)
import jax
import jax.numpy as jnp
from jax.experimental import pallas as pl


def kernel(flat_latents, label, embedding):
    raise NotImplementedError("write your pallas kernel here")


# speedup vs baseline: 174.0228x; 174.0228x over previous
import jax
import jax.numpy as jnp
from jax.experimental import pallas as pl
from jax.experimental.pallas import tpu as pltpu

_K = 512
_D = 256
_B = 576
_BETA = 0.25
_TEMP = 0.1


def _vq_kernel(x_ref, lab_ref, e_ref, q_ref, loss_ref):
    x = x_ref[...]
    e = e_ref[...]
    lab = lab_ref[...]

    ohT = (jax.lax.broadcasted_iota(jnp.int32, (_K, _B), 0) == lab).astype(
        jnp.float32)
    p = jax.lax.dot_general(
        ohT, e, (((0,), (0,)), ((), ())),
        preferred_element_type=jnp.float32,
        precision=jax.lax.Precision.HIGHEST)
    q_ref[...] = p

    mse = jnp.mean((p - x) ** 2)

    qn = x * jax.lax.rsqrt(jnp.sum(x * x, axis=1, keepdims=True))
    en = e * jax.lax.rsqrt(jnp.sum(e * e, axis=1, keepdims=True))
    pn = p * jax.lax.rsqrt(jnp.sum(p * p, axis=1, keepdims=True))

    logits = jax.lax.dot_general(
        qn, en, (((1,), (1,)), ((), ())),
        preferred_element_type=jnp.float32,
        precision=jax.lax.Precision.HIGHEST) * (1.0 / _TEMP)
    pos = jnp.sum(qn * pn, axis=1) * (1.0 / _TEMP)

    m = jnp.max(logits, axis=1)
    lse = jnp.log(jnp.sum(jnp.exp(logits - m[:, None]), axis=1)) + m
    infonce = jnp.mean(lse - pos)

    loss_ref[...] = jnp.reshape(mse * (1.0 + _BETA) + infonce, (1, 1))


def kernel(flat_latents, label, embedding):
    lab2d = label.reshape(1, _B)
    q, loss = pl.pallas_call(
        _vq_kernel,
        out_shape=(
            jax.ShapeDtypeStruct((_B, _D), jnp.float32),
            jax.ShapeDtypeStruct((1, 1), jnp.float32),
        ),
    )(flat_latents, lab2d, embedding)
    return q, loss.reshape(())
